# baseline (speedup 1.0000x reference)
import functools

import jax
import jax.numpy as jnp
from jax import lax
from jax.experimental import pallas as pl
from jax.experimental.pallas import tpu as pltpu
from jax.experimental.pallas import tpu_sc as plsc

N = 10000
E = 320000
R = 8

NC = 2
NS = 16
NW = NC * NS

K = 128
EPW = 10240
STEPS = EPW // K
GSTEPS = STEPS + 2
E_PAD = EPW * NW
NP = 10112
ROWS = NP // NS


def _mm1_kernel(x_ref, v_ref, comb_ref, loopw_ref, y_ref, loop_ref):
    w = jnp.sum(comb_ref[...][:, :, None, None] * v_ref[...][None], axis=1)
    x = x_ref[...]
    for r in range(R):
        y_ref[r] = jnp.dot(x, w[r], preferred_element_type=jnp.float32)
    loop_ref[...] = jnp.dot(x, loopw_ref[...], preferred_element_type=jnp.float32)


def _mm2_kernel(p_ref, loop1_ref, b1_ref, v_ref, comb_ref, loopw_ref,
                y_ref, loop_ref):
    h = p_ref[0] + p_ref[1] + loop1_ref[...] + b1_ref[...]
    h = jnp.maximum(h, 0.0)
    w = jnp.sum(comb_ref[...][:, :, None, None] * v_ref[...][None], axis=1)
    for r in range(R):
        y_ref[r] = jnp.dot(h, w[r], preferred_element_type=jnp.float32)
    loop_ref[...] = jnp.dot(h, loopw_ref[...], preferred_element_type=jnp.float32)


def _final_kernel(p_ref, loop2_ref, b2_ref, out_ref):
    i = pl.program_id(0)

    @pl.when(i == 0)
    def _():
        out_ref[...] = jnp.zeros_like(out_ref)

    h = p_ref[0] + p_ref[1] + loop2_ref[...] + b2_ref[...]
    h = jnp.maximum(h, 0.0)
    out_ref[...] += jnp.sum(h, axis=0, keepdims=True) * (1.0 / N)


def _make_sc_scatter(d):
    mesh = plsc.VectorSubcoreMesh(core_axis_name="c", subcore_axis_name="s")

    @functools.partial(
        pl.kernel,
        mesh=mesh,
        out_type=jax.ShapeDtypeStruct((NC, NP, d), jnp.float32),
        scratch_types=[
            pltpu.VMEM((GSTEPS, K), jnp.int32),
            pltpu.VMEM((STEPS, K), jnp.int32),
            pltpu.VMEM((K, d), jnp.float32),
            pltpu.VMEM((K, d), jnp.float32),
            pltpu.VMEM_SHARED((NP, d), jnp.float32),
            pltpu.SemaphoreType.DMA,
            pltpu.SemaphoreType.DMA,
        ],
        compiler_params=pltpu.CompilerParams(use_tc_tiling_on_sc=False),
    )
    def sc_kernel(gidx_hbm, didx_hbm, zeros_hbm, table_hbm, out_hbm,
                  gidx_v, didx_v, rows_a, rows_b, acc_sh, sem_a, sem_b):
        cid = lax.axis_index("c")
        sid = lax.axis_index("s")
        wid = sid * NC + cid

        pltpu.sync_copy(zeros_hbm.at[pl.ds(sid * ROWS, ROWS)],
                        acc_sh.at[pl.ds(sid * ROWS, ROWS)])
        pltpu.sync_copy(gidx_hbm.at[wid], gidx_v)
        pltpu.sync_copy(didx_hbm.at[wid], didx_v)
        plsc.subcore_barrier()

        pltpu.async_copy(table_hbm.at[gidx_v.at[0]], rows_a, sem_a)

        def body(i, carry):
            j = 2 * i
            pltpu.async_copy(table_hbm.at[gidx_v.at[j + 1]], rows_b, sem_b)
            pltpu.make_async_copy(table_hbm.at[gidx_v.at[j]],
                                  rows_a, sem_a).wait()
            pltpu.sync_copy(rows_a, acc_sh.at[didx_v.at[j]], add=True)
            pltpu.async_copy(table_hbm.at[gidx_v.at[j + 2]], rows_a, sem_a)
            pltpu.make_async_copy(table_hbm.at[gidx_v.at[j + 1]],
                                  rows_b, sem_b).wait()
            pltpu.sync_copy(rows_b, acc_sh.at[didx_v.at[j + 1]], add=True)
            return carry

        lax.fori_loop(0, STEPS // 2, body, 0)
        pltpu.make_async_copy(table_hbm.at[gidx_v.at[STEPS]],
                              rows_a, sem_a).wait()
        plsc.subcore_barrier()

        pltpu.sync_copy(acc_sh.at[pl.ds(sid * ROWS, ROWS)],
                        out_hbm.at[cid, pl.ds(sid * ROWS, ROWS)])

    return sc_kernel


_sc_scatter_64 = _make_sc_scatter(64)
_sc_scatter_16 = _make_sc_scatter(16)

_BN = 400
_GRID = N // _BN


def _mm1(x, v1, comb1, loop_w1):
    h = v1.shape[-1]
    return pl.pallas_call(
        _mm1_kernel,
        grid=(_GRID,),
        in_specs=[
            pl.BlockSpec((_BN, x.shape[1]), lambda i: (i, 0)),
            pl.BlockSpec(v1.shape, lambda i: (0, 0, 0)),
            pl.BlockSpec(comb1.shape, lambda i: (0, 0)),
            pl.BlockSpec(loop_w1.shape, lambda i: (0, 0)),
        ],
        out_specs=[
            pl.BlockSpec((R, _BN, h), lambda i: (0, i, 0)),
            pl.BlockSpec((_BN, h), lambda i: (i, 0)),
        ],
        out_shape=[
            jax.ShapeDtypeStruct((R, N, h), jnp.float32),
            jax.ShapeDtypeStruct((N, h), jnp.float32),
        ],
    )(x, v1, comb1, loop_w1)


def _mm2(parts, loop1, b1, v2, comb2, loop_w2):
    h = parts.shape[-1]
    c = v2.shape[-1]
    return pl.pallas_call(
        _mm2_kernel,
        grid=(_GRID,),
        in_specs=[
            pl.BlockSpec((NC, _BN, h), lambda i: (0, i, 0)),
            pl.BlockSpec((_BN, h), lambda i: (i, 0)),
            pl.BlockSpec((1, h), lambda i: (0, 0)),
            pl.BlockSpec(v2.shape, lambda i: (0, 0, 0)),
            pl.BlockSpec(comb2.shape, lambda i: (0, 0)),
            pl.BlockSpec(loop_w2.shape, lambda i: (0, 0)),
        ],
        out_specs=[
            pl.BlockSpec((R, _BN, c), lambda i: (0, i, 0)),
            pl.BlockSpec((_BN, c), lambda i: (i, 0)),
        ],
        out_shape=[
            jax.ShapeDtypeStruct((R, N, c), jnp.float32),
            jax.ShapeDtypeStruct((N, c), jnp.float32),
        ],
    )(parts, loop1, b1, v2, comb2, loop_w2)


def _final(parts, loop2, b2):
    c = parts.shape[-1]
    return pl.pallas_call(
        _final_kernel,
        grid=(_GRID,),
        in_specs=[
            pl.BlockSpec((NC, _BN, c), lambda i: (0, i, 0)),
            pl.BlockSpec((_BN, c), lambda i: (i, 0)),
            pl.BlockSpec((1, c), lambda i: (0, 0)),
        ],
        out_specs=pl.BlockSpec((1, c), lambda i: (0, 0)),
        out_shape=jax.ShapeDtypeStruct((1, c), jnp.float32),
    )(parts, loop2, b2)


def kernel(in_feat, edge_index, etypes, V1, comb1, loop_w1, b1,
           V2, comb2, loop_w2, b2):
    src = edge_index[0]
    dst = edge_index[1]
    et = etypes.reshape(-1)

    gidx = (et * N + src).astype(jnp.int32)
    didx = dst.astype(jnp.int32)
    pad = E_PAD - E
    gidx = jnp.concatenate([gidx, jnp.zeros((pad,), jnp.int32)])
    didx = jnp.concatenate([didx, jnp.full((pad,), N, jnp.int32)])
    gidx = gidx.reshape(NW, STEPS, K)
    gidx = jnp.concatenate(
        [gidx, jnp.zeros((NW, GSTEPS - STEPS, K), jnp.int32)], axis=1)
    didx = didx.reshape(NW, STEPS, K)

    h = V1.shape[-1]
    c = V2.shape[-1]
    zeros_h = jnp.zeros((NP, h), jnp.float32)
    zeros_c = jnp.zeros((NP, c), jnp.float32)

    y1, loop1 = _mm1(in_feat, V1, comb1, loop_w1)
    parts1 = _sc_scatter_64(gidx, didx, zeros_h, y1.reshape(R * N, h))
    parts1 = parts1[:, :N]

    y2, loop2 = _mm2(parts1, loop1, b1.reshape(1, h), V2, comb2, loop_w2)
    parts2 = _sc_scatter_16(gidx, didx, zeros_c, y2.reshape(R * N, c))
    parts2 = parts2[:, :N]

    return _final(parts2, loop2, b2.reshape(1, c))


# speedup vs baseline: 1.2202x; 1.2202x over previous
import functools

import jax
import jax.numpy as jnp
from jax import lax
from jax.experimental import pallas as pl
from jax.experimental.pallas import tpu as pltpu
from jax.experimental.pallas import tpu_sc as plsc

N = 10000
E = 320000
R = 8

NC = 2
NS = 16
NW = NC * NS

K = 128
SPS = 160
E_PAD = SPS * NS * K
TOT_ROWS = E_PAD // K
NP = 10112
ROWS = NP // NS


def _mm1_kernel(x_ref, v_ref, comb_ref, loopw_ref, y_ref, loop_ref):
    w = jnp.sum(comb_ref[...][:, :, None, None] * v_ref[...][None], axis=1)
    x = x_ref[...]
    for r in range(R):
        y_ref[r] = jnp.dot(x, w[r], preferred_element_type=jnp.float32)
    loop_ref[...] = jnp.dot(x, loopw_ref[...], preferred_element_type=jnp.float32)


def _mm2_kernel(p_ref, loop1_ref, b1_ref, v_ref, comb_ref, loopw_ref,
                y_ref, loop_ref):
    h = p_ref[0] + p_ref[1] + loop1_ref[...] + b1_ref[...]
    h = jnp.maximum(h, 0.0)
    w = jnp.sum(comb_ref[...][:, :, None, None] * v_ref[...][None], axis=1)
    for r in range(R):
        y_ref[r] = jnp.dot(h, w[r], preferred_element_type=jnp.float32)
    loop_ref[...] = jnp.dot(h, loopw_ref[...], preferred_element_type=jnp.float32)


def _final_kernel(p_ref, loop2_ref, b2_ref, out_ref):
    i = pl.program_id(0)

    @pl.when(i == 0)
    def _():
        out_ref[...] = jnp.zeros_like(out_ref)

    h = p_ref[0] + p_ref[1] + loop2_ref[...] + b2_ref[...]
    h = jnp.maximum(h, 0.0)
    out_ref[...] += jnp.sum(h, axis=0, keepdims=True) * (1.0 / N)


def _make_sc_scatter(d, s0):
    s1 = SPS - s0
    smax = max(s0, s1)
    mesh = plsc.VectorSubcoreMesh(core_axis_name="c", subcore_axis_name="s")

    @functools.partial(
        pl.kernel,
        mesh=mesh,
        out_type=jax.ShapeDtypeStruct((NC, NP, d), jnp.float32),
        scratch_types=[
            pltpu.VMEM((smax, K), jnp.int32),
            pltpu.VMEM((smax, K), jnp.int32),
            pltpu.VMEM((K, d), jnp.float32),
            pltpu.VMEM_SHARED((NP, d), jnp.float32),
            pltpu.SemaphoreType.DMA,
        ],
        compiler_params=pltpu.CompilerParams(use_tc_tiling_on_sc=False),
    )
    def sc_kernel(gidx_hbm, didx_hbm, zeros_hbm, table_hbm, out_hbm,
                  gidx_v, didx_v, rows_v, acc_sh, sem):
        cid = lax.axis_index("c")
        sid = lax.axis_index("s")
        base = sid * SPS + cid * s0
        steps = lax.select(cid == 0, s0, s1)

        pltpu.sync_copy(zeros_hbm.at[pl.ds(sid * ROWS, ROWS)],
                        acc_sh.at[pl.ds(sid * ROWS, ROWS)])
        pltpu.sync_copy(gidx_hbm.at[pl.ds(base, smax)], gidx_v)
        pltpu.sync_copy(didx_hbm.at[pl.ds(base, smax)], didx_v)
        plsc.subcore_barrier()

        def body(j, carry):
            pltpu.async_copy(table_hbm.at[gidx_v.at[j]], rows_v, sem).wait()
            pltpu.sync_copy(rows_v, acc_sh.at[didx_v.at[j]], add=True)
            return carry

        lax.fori_loop(0, steps, body, 0)
        plsc.subcore_barrier()

        pltpu.sync_copy(acc_sh.at[pl.ds(sid * ROWS, ROWS)],
                        out_hbm.at[cid, pl.ds(sid * ROWS, ROWS)])

    return sc_kernel


_EXTRA = 128
_sc_scatter_64 = _make_sc_scatter(64, 110)
_sc_scatter_16 = _make_sc_scatter(16, 92)

_BN = 400
_GRID = N // _BN


def _mm1(x, v1, comb1, loop_w1):
    h = v1.shape[-1]
    return pl.pallas_call(
        _mm1_kernel,
        grid=(_GRID,),
        in_specs=[
            pl.BlockSpec((_BN, x.shape[1]), lambda i: (i, 0)),
            pl.BlockSpec(v1.shape, lambda i: (0, 0, 0)),
            pl.BlockSpec(comb1.shape, lambda i: (0, 0)),
            pl.BlockSpec(loop_w1.shape, lambda i: (0, 0)),
        ],
        out_specs=[
            pl.BlockSpec((R, _BN, h), lambda i: (0, i, 0)),
            pl.BlockSpec((_BN, h), lambda i: (i, 0)),
        ],
        out_shape=[
            jax.ShapeDtypeStruct((R, N, h), jnp.float32),
            jax.ShapeDtypeStruct((N, h), jnp.float32),
        ],
    )(x, v1, comb1, loop_w1)


def _mm2(parts, loop1, b1, v2, comb2, loop_w2):
    h = parts.shape[-1]
    c = v2.shape[-1]
    return pl.pallas_call(
        _mm2_kernel,
        grid=(_GRID,),
        in_specs=[
            pl.BlockSpec((NC, _BN, h), lambda i: (0, i, 0)),
            pl.BlockSpec((_BN, h), lambda i: (i, 0)),
            pl.BlockSpec((1, h), lambda i: (0, 0)),
            pl.BlockSpec(v2.shape, lambda i: (0, 0, 0)),
            pl.BlockSpec(comb2.shape, lambda i: (0, 0)),
            pl.BlockSpec(loop_w2.shape, lambda i: (0, 0)),
        ],
        out_specs=[
            pl.BlockSpec((R, _BN, c), lambda i: (0, i, 0)),
            pl.BlockSpec((_BN, c), lambda i: (i, 0)),
        ],
        out_shape=[
            jax.ShapeDtypeStruct((R, N, c), jnp.float32),
            jax.ShapeDtypeStruct((N, c), jnp.float32),
        ],
    )(parts, loop1, b1, v2, comb2, loop_w2)


def _final(parts, loop2, b2):
    c = parts.shape[-1]
    return pl.pallas_call(
        _final_kernel,
        grid=(_GRID,),
        in_specs=[
            pl.BlockSpec((NC, _BN, c), lambda i: (0, i, 0)),
            pl.BlockSpec((_BN, c), lambda i: (i, 0)),
            pl.BlockSpec((1, c), lambda i: (0, 0)),
        ],
        out_specs=pl.BlockSpec((1, c), lambda i: (0, 0)),
        out_shape=jax.ShapeDtypeStruct((1, c), jnp.float32),
    )(parts, loop2, b2)


def kernel(in_feat, edge_index, etypes, V1, comb1, loop_w1, b1,
           V2, comb2, loop_w2, b2):
    src = edge_index[0]
    dst = edge_index[1]
    et = etypes.reshape(-1)

    gidx = (et * N + src).astype(jnp.int32)
    didx = dst.astype(jnp.int32)
    pad = E_PAD - E
    padx = pad + _EXTRA * K
    gidx = jnp.concatenate([gidx, jnp.zeros((padx,), jnp.int32)])
    didx = jnp.concatenate([didx, jnp.full((padx,), N, jnp.int32)])
    gidx = gidx.reshape(TOT_ROWS + _EXTRA, K)
    didx = didx.reshape(TOT_ROWS + _EXTRA, K)

    h = V1.shape[-1]
    c = V2.shape[-1]
    zeros_h = jnp.zeros((NP, h), jnp.float32)
    zeros_c = jnp.zeros((NP, c), jnp.float32)

    y1, loop1 = _mm1(in_feat, V1, comb1, loop_w1)
    parts1 = _sc_scatter_64(gidx, didx, zeros_h, y1.reshape(R * N, h))
    parts1 = parts1[:, :N]

    y2, loop2 = _mm2(parts1, loop1, b1.reshape(1, h), V2, comb2, loop_w2)
    parts2 = _sc_scatter_16(gidx, didx, zeros_c, y2.reshape(R * N, c))
    parts2 = parts2[:, :N]

    return _final(parts2, loop2, b2.reshape(1, c))


# speedup vs baseline: 1.2876x; 1.0553x over previous
import functools

import jax
import jax.numpy as jnp
from jax import lax
from jax.experimental import pallas as pl
from jax.experimental.pallas import tpu as pltpu
from jax.experimental.pallas import tpu_sc as plsc

N = 10000
E = 320000
R = 8

NC = 2
NS = 16
NW = NC * NS

K = 128
SPS = 160
E_PAD = SPS * NS * K
TOT_ROWS = E_PAD // K
NP = 10112
ROWS = NP // NS


def _mm1_kernel(x_ref, v_ref, comb_ref, loopw_ref, y_ref, loop_ref):
    w = jnp.sum(comb_ref[...][:, :, None, None] * v_ref[...][None], axis=1)
    x = x_ref[...]
    for r in range(R):
        y_ref[r] = jnp.dot(x, w[r], preferred_element_type=jnp.float32)
    loop_ref[...] = jnp.dot(x, loopw_ref[...], preferred_element_type=jnp.float32)


def _mm2_kernel(p_ref, loop1_ref, b1_ref, v_ref, comb_ref, loopw_ref,
                y_ref, loop_ref):
    h = p_ref[0] + p_ref[1] + loop1_ref[...] + b1_ref[...]
    h = jnp.maximum(h, 0.0)
    w = jnp.sum(comb_ref[...][:, :, None, None] * v_ref[...][None], axis=1)
    for r in range(R):
        y_ref[r] = jnp.dot(h, w[r], preferred_element_type=jnp.float32)
    loop_ref[...] = jnp.dot(h, loopw_ref[...], preferred_element_type=jnp.float32)


def _final_kernel(p_ref, loop2_ref, b2_ref, out_ref):
    i = pl.program_id(0)

    @pl.when(i == 0)
    def _():
        out_ref[...] = jnp.zeros_like(out_ref)

    h = p_ref[0] + p_ref[1] + loop2_ref[...] + b2_ref[...]
    h = jnp.maximum(h, 0.0)
    out_ref[...] += jnp.sum(h, axis=0, keepdims=True) * (1.0 / N)


def _make_sc_scatter(d, s0):
    s1 = SPS - s0
    smax = max(s0, s1)
    mesh = plsc.VectorSubcoreMesh(core_axis_name="c", subcore_axis_name="s")

    @functools.partial(
        pl.kernel,
        mesh=mesh,
        out_type=jax.ShapeDtypeStruct((NC, NP, d), jnp.float32),
        scratch_types=[
            pltpu.VMEM((smax, K), jnp.int32),
            pltpu.VMEM((smax, K), jnp.int32),
            pltpu.VMEM((K, d), jnp.float32),
            pltpu.VMEM_SHARED((NP, d), jnp.float32),
            pltpu.SemaphoreType.DMA,
        ],
        compiler_params=pltpu.CompilerParams(use_tc_tiling_on_sc=False),
    )
    def sc_kernel(gidx_hbm, didx_hbm, zeros_hbm, table_hbm, out_hbm,
                  gidx_v, didx_v, rows_v, acc_sh, sem):
        cid = lax.axis_index("c")
        sid = lax.axis_index("s")
        base = sid * SPS + cid * s0
        steps = lax.select(cid == 0, s0, s1)

        pltpu.sync_copy(zeros_hbm.at[pl.ds(sid * ROWS, ROWS)],
                        acc_sh.at[pl.ds(sid * ROWS, ROWS)])
        pltpu.sync_copy(gidx_hbm.at[pl.ds(base, smax)], gidx_v)
        pltpu.sync_copy(didx_hbm.at[pl.ds(base, smax)], didx_v)
        plsc.subcore_barrier()

        def body(j, carry):
            pltpu.async_copy(table_hbm.at[gidx_v.at[j]], rows_v, sem).wait()
            pltpu.sync_copy(rows_v, acc_sh.at[didx_v.at[j]], add=True)
            return carry

        lax.fori_loop(0, steps, body, 0)
        plsc.subcore_barrier()

        pltpu.sync_copy(acc_sh.at[pl.ds(sid * ROWS, ROWS)],
                        out_hbm.at[cid, pl.ds(sid * ROWS, ROWS)])

    return sc_kernel


_EXTRA = 128
_sc_scatter_64 = _make_sc_scatter(64, 113)
_sc_scatter_16 = _make_sc_scatter(16, 99)

_BN = 400
_GRID = N // _BN


def _mm1(x, v1, comb1, loop_w1):
    h = v1.shape[-1]
    return pl.pallas_call(
        _mm1_kernel,
        grid=(_GRID,),
        in_specs=[
            pl.BlockSpec((_BN, x.shape[1]), lambda i: (i, 0)),
            pl.BlockSpec(v1.shape, lambda i: (0, 0, 0)),
            pl.BlockSpec(comb1.shape, lambda i: (0, 0)),
            pl.BlockSpec(loop_w1.shape, lambda i: (0, 0)),
        ],
        out_specs=[
            pl.BlockSpec((R, _BN, h), lambda i: (0, i, 0)),
            pl.BlockSpec((_BN, h), lambda i: (i, 0)),
        ],
        out_shape=[
            jax.ShapeDtypeStruct((R, N, h), jnp.float32),
            jax.ShapeDtypeStruct((N, h), jnp.float32),
        ],
    )(x, v1, comb1, loop_w1)


def _mm2(parts, loop1, b1, v2, comb2, loop_w2):
    h = parts.shape[-1]
    c = v2.shape[-1]
    return pl.pallas_call(
        _mm2_kernel,
        grid=(_GRID,),
        in_specs=[
            pl.BlockSpec((NC, _BN, h), lambda i: (0, i, 0)),
            pl.BlockSpec((_BN, h), lambda i: (i, 0)),
            pl.BlockSpec((1, h), lambda i: (0, 0)),
            pl.BlockSpec(v2.shape, lambda i: (0, 0, 0)),
            pl.BlockSpec(comb2.shape, lambda i: (0, 0)),
            pl.BlockSpec(loop_w2.shape, lambda i: (0, 0)),
        ],
        out_specs=[
            pl.BlockSpec((R, _BN, c), lambda i: (0, i, 0)),
            pl.BlockSpec((_BN, c), lambda i: (i, 0)),
        ],
        out_shape=[
            jax.ShapeDtypeStruct((R, N, c), jnp.float32),
            jax.ShapeDtypeStruct((N, c), jnp.float32),
        ],
    )(parts, loop1, b1, v2, comb2, loop_w2)


def _final(parts, loop2, b2):
    c = parts.shape[-1]
    return pl.pallas_call(
        _final_kernel,
        grid=(_GRID,),
        in_specs=[
            pl.BlockSpec((NC, _BN, c), lambda i: (0, i, 0)),
            pl.BlockSpec((_BN, c), lambda i: (i, 0)),
            pl.BlockSpec((1, c), lambda i: (0, 0)),
        ],
        out_specs=pl.BlockSpec((1, c), lambda i: (0, 0)),
        out_shape=jax.ShapeDtypeStruct((1, c), jnp.float32),
    )(parts, loop2, b2)


def kernel(in_feat, edge_index, etypes, V1, comb1, loop_w1, b1,
           V2, comb2, loop_w2, b2):
    src = edge_index[0]
    dst = edge_index[1]
    et = etypes.reshape(-1)

    gidx = (et * N + src).astype(jnp.int32)
    didx = dst.astype(jnp.int32)
    pad = E_PAD - E
    padx = pad + _EXTRA * K
    gidx = jnp.concatenate([gidx, jnp.zeros((padx,), jnp.int32)])
    didx = jnp.concatenate([didx, jnp.full((padx,), N, jnp.int32)])
    gidx = gidx.reshape(TOT_ROWS + _EXTRA, K)
    didx = didx.reshape(TOT_ROWS + _EXTRA, K)

    h = V1.shape[-1]
    c = V2.shape[-1]
    zeros_h = jnp.zeros((NP, h), jnp.float32)
    zeros_c = jnp.zeros((NP, c), jnp.float32)

    y1, loop1 = _mm1(in_feat, V1, comb1, loop_w1)
    parts1 = _sc_scatter_64(gidx, didx, zeros_h, y1.reshape(R * N, h))

    y2, loop2 = _mm2(parts1, loop1, b1.reshape(1, h), V2, comb2, loop_w2)
    parts2 = _sc_scatter_16(gidx, didx, zeros_c, y2.reshape(R * N, c))

    return _final(parts2, loop2, b2.reshape(1, c))


# speedup vs baseline: 1.5020x; 1.1665x over previous
import functools

import jax
import jax.numpy as jnp
from jax import lax
from jax.experimental import pallas as pl
from jax.experimental.pallas import tpu as pltpu
from jax.experimental.pallas import tpu_sc as plsc

N = 10000
E = 320000
R = 8

NC = 2
NS = 16
NW = NC * NS

K = 128
SPS = 160
E_PAD = SPS * NS * K
TOT_ROWS = E_PAD // K
NP = 10112
ROWS = NP // NS


def _mm1_kernel(x_ref, v_ref, comb_ref, loopw_ref, y_ref, loop_ref):
    w = jnp.sum(comb_ref[...][:, :, None, None] * v_ref[...][None], axis=1)
    x = x_ref[...]
    for p in range(R // 2):
        wp = jnp.concatenate([w[2 * p], w[2 * p + 1]], axis=-1)
        y_ref[p] = jnp.dot(x, wp, preferred_element_type=jnp.float32)
    loop_ref[...] = jnp.dot(x, loopw_ref[...], preferred_element_type=jnp.float32)


def _mm2_kernel(p_ref, loop1_ref, b1_ref, v_ref, comb_ref, loopw_ref,
                y_ref, loop_ref):
    h = p_ref[0] + p_ref[1] + loop1_ref[...] + b1_ref[...]
    h = jnp.maximum(h, 0.0)
    w = jnp.sum(comb_ref[...][:, :, None, None] * v_ref[...][None], axis=1)
    wcat = jnp.concatenate([w[r] for r in range(R)], axis=-1)
    y_ref[...] = jnp.dot(h, wcat, preferred_element_type=jnp.float32)
    loop_ref[...] = jnp.dot(h, loopw_ref[...], preferred_element_type=jnp.float32)


def _final_kernel(p_ref, loop2_ref, b2_ref, out_ref):
    i = pl.program_id(0)

    @pl.when(i == 0)
    def _():
        out_ref[...] = jnp.zeros_like(out_ref)

    h = p_ref[0] + p_ref[1] + loop2_ref[...] + b2_ref[...]
    h = jnp.maximum(h, 0.0)
    out_ref[...] += jnp.sum(h, axis=0, keepdims=True) * (1.0 / N)


def _make_sc_scatter(d, s0):
    s1 = SPS - s0
    smax = max(s0, s1)
    mesh = plsc.VectorSubcoreMesh(core_axis_name="c", subcore_axis_name="s")

    @functools.partial(
        pl.kernel,
        mesh=mesh,
        out_type=jax.ShapeDtypeStruct((NC, NP, d), jnp.float32),
        scratch_types=[
            pltpu.VMEM((smax, K), jnp.int32),
            pltpu.VMEM((smax, K), jnp.int32),
            pltpu.VMEM((K, d), jnp.float32),
            pltpu.VMEM_SHARED((NP, d), jnp.float32),
            pltpu.SemaphoreType.DMA,
        ],
        compiler_params=pltpu.CompilerParams(use_tc_tiling_on_sc=False),
    )
    def sc_kernel(gidx_hbm, didx_hbm, zeros_hbm, table_hbm, out_hbm,
                  gidx_v, didx_v, rows_v, acc_sh, sem):
        cid = lax.axis_index("c")
        sid = lax.axis_index("s")
        base = sid * SPS + cid * s0
        steps = lax.select(cid == 0, s0, s1)

        pltpu.sync_copy(zeros_hbm.at[pl.ds(sid * ROWS, ROWS)],
                        acc_sh.at[pl.ds(sid * ROWS, ROWS)])
        pltpu.sync_copy(gidx_hbm.at[pl.ds(base, smax)], gidx_v)
        pltpu.sync_copy(didx_hbm.at[pl.ds(base, smax)], didx_v)
        plsc.subcore_barrier()

        def body(j, carry):
            pltpu.async_copy(table_hbm.at[gidx_v.at[j]], rows_v, sem).wait()
            pltpu.sync_copy(rows_v, acc_sh.at[didx_v.at[j]], add=True)
            return carry

        lax.fori_loop(0, steps, body, 0)
        plsc.subcore_barrier()

        pltpu.sync_copy(acc_sh.at[pl.ds(sid * ROWS, ROWS)],
                        out_hbm.at[cid, pl.ds(sid * ROWS, ROWS)])

    return sc_kernel


_EXTRA = 128
_sc_scatter_64 = _make_sc_scatter(64, 113)
_sc_scatter_16 = _make_sc_scatter(16, 99)

_BN = 400
_GRID = N // _BN


def _mm1(x, v1, comb1, loop_w1):
    h = v1.shape[-1]
    return pl.pallas_call(
        _mm1_kernel,
        grid=(_GRID,),
        in_specs=[
            pl.BlockSpec((_BN, x.shape[1]), lambda i: (i, 0)),
            pl.BlockSpec(v1.shape, lambda i: (0, 0, 0)),
            pl.BlockSpec(comb1.shape, lambda i: (0, 0)),
            pl.BlockSpec(loop_w1.shape, lambda i: (0, 0)),
        ],
        out_specs=[
            pl.BlockSpec((R // 2, _BN, 128), lambda i: (0, i, 0)),
            pl.BlockSpec((_BN, h), lambda i: (i, 0)),
        ],
        out_shape=[
            jax.ShapeDtypeStruct((R // 2, N, 128), jnp.float32),
            jax.ShapeDtypeStruct((N, h), jnp.float32),
        ],
    )(x, v1, comb1, loop_w1)


def _mm2(parts, loop1, b1, v2, comb2, loop_w2):
    h = parts.shape[-1]
    c = v2.shape[-1]
    return pl.pallas_call(
        _mm2_kernel,
        grid=(_GRID,),
        in_specs=[
            pl.BlockSpec((NC, _BN, h), lambda i: (0, i, 0)),
            pl.BlockSpec((_BN, h), lambda i: (i, 0)),
            pl.BlockSpec((1, h), lambda i: (0, 0)),
            pl.BlockSpec(v2.shape, lambda i: (0, 0, 0)),
            pl.BlockSpec(comb2.shape, lambda i: (0, 0)),
            pl.BlockSpec(loop_w2.shape, lambda i: (0, 0)),
        ],
        out_specs=[
            pl.BlockSpec((_BN, 128), lambda i: (i, 0)),
            pl.BlockSpec((_BN, c), lambda i: (i, 0)),
        ],
        out_shape=[
            jax.ShapeDtypeStruct((N, 128), jnp.float32),
            jax.ShapeDtypeStruct((N, c), jnp.float32),
        ],
    )(parts, loop1, b1, v2, comb2, loop_w2)


def _final(parts, loop2, b2):
    c = parts.shape[-1]
    return pl.pallas_call(
        _final_kernel,
        grid=(_GRID,),
        in_specs=[
            pl.BlockSpec((NC, _BN, c), lambda i: (0, i, 0)),
            pl.BlockSpec((_BN, c), lambda i: (i, 0)),
            pl.BlockSpec((1, c), lambda i: (0, 0)),
        ],
        out_specs=pl.BlockSpec((1, c), lambda i: (0, 0)),
        out_shape=jax.ShapeDtypeStruct((1, c), jnp.float32),
    )(parts, loop2, b2)


def kernel(in_feat, edge_index, etypes, V1, comb1, loop_w1, b1,
           V2, comb2, loop_w2, b2):
    src = edge_index[0]
    dst = edge_index[1]
    et = etypes.reshape(-1)

    gidx1 = ((et >> 1) * (2 * N) + 2 * src + (et & 1)).astype(jnp.int32)
    gidx2 = (src * R + et).astype(jnp.int32)
    didx = dst.astype(jnp.int32)
    pad = E_PAD - E
    padx = pad + _EXTRA * K
    zpad = jnp.zeros((padx,), jnp.int32)
    gidx1 = jnp.concatenate([gidx1, zpad]).reshape(TOT_ROWS + _EXTRA, K)
    gidx2 = jnp.concatenate([gidx2, zpad]).reshape(TOT_ROWS + _EXTRA, K)
    didx = jnp.concatenate([didx, jnp.full((padx,), N, jnp.int32)])
    didx = didx.reshape(TOT_ROWS + _EXTRA, K)

    h = V1.shape[-1]
    c = V2.shape[-1]
    zeros_h = jnp.zeros((NP, h), jnp.float32)
    zeros_c = jnp.zeros((NP, c), jnp.float32)

    y1, loop1 = _mm1(in_feat, V1, comb1, loop_w1)
    parts1 = _sc_scatter_64(gidx1, didx, zeros_h, y1.reshape(R * N, h))

    y2, loop2 = _mm2(parts1, loop1, b1.reshape(1, h), V2, comb2, loop_w2)
    parts2 = _sc_scatter_16(gidx2, didx, zeros_c, y2.reshape(R * N, c))

    return _final(parts2, loop2, b2.reshape(1, c))


# speedup vs baseline: 1.6647x; 1.1083x over previous
import functools

import jax
import jax.numpy as jnp
from jax import lax
from jax.experimental import pallas as pl
from jax.experimental.pallas import tpu as pltpu
from jax.experimental.pallas import tpu_sc as plsc

N = 10000
E = 320000
R = 8

NC = 2
NS = 16
NW = NC * NS

K = 128
SPS = 160
E_PAD = SPS * NS * K
TOT_ROWS = E_PAD // K
NP = 10112
ROWS = NP // NS


def _mm1_kernel(x_ref, v_ref, comb_ref, loopw_ref, y_ref, loop_ref):
    w = jnp.sum(comb_ref[...][:, :, None, None] * v_ref[...][None], axis=1)
    x = x_ref[...]
    for p in range(R // 2):
        wp = jnp.concatenate([w[2 * p], w[2 * p + 1]], axis=-1)
        y_ref[p] = jnp.dot(x, wp, preferred_element_type=jnp.float32)
    loop_ref[...] = jnp.dot(x, loopw_ref[...], preferred_element_type=jnp.float32)


def _mm2_kernel(p_ref, loop1_ref, b1_ref, v_ref, comb_ref, loopw_ref,
                y_ref, loop_ref):
    h = p_ref[0] + p_ref[1] + loop1_ref[...] + b1_ref[...]
    h = jnp.maximum(h, 0.0)
    w = jnp.sum(comb_ref[...][:, :, None, None] * v_ref[...][None], axis=1)
    wcat = jnp.concatenate([w[r] for r in range(R)], axis=-1)
    y_ref[...] = jnp.dot(h, wcat, preferred_element_type=jnp.float32)
    loop_ref[...] = jnp.dot(h, loopw_ref[...], preferred_element_type=jnp.float32)


def _final_kernel(p_ref, loop2_ref, b2_ref, out_ref):
    i = pl.program_id(0)

    @pl.when(i == 0)
    def _():
        out_ref[...] = jnp.zeros_like(out_ref)

    h = p_ref[0] + p_ref[1] + loop2_ref[...] + b2_ref[...]
    h = jnp.maximum(h, 0.0)
    out_ref[...] += jnp.sum(h, axis=0, keepdims=True) * (1.0 / N)


def _make_sc_scatter(d, s0, pipelined=False):
    s1 = SPS - s0
    smax = max(s0, s1)
    smax_g = smax + 2 if pipelined else smax
    mesh = plsc.VectorSubcoreMesh(core_axis_name="c", subcore_axis_name="s")

    @functools.partial(
        pl.kernel,
        mesh=mesh,
        out_type=jax.ShapeDtypeStruct((NC, NP, d), jnp.float32),
        scratch_types=[
            pltpu.VMEM((smax_g, K), jnp.int32),
            pltpu.VMEM((smax, K), jnp.int32),
            pltpu.VMEM((K, d), jnp.float32),
            pltpu.VMEM((K, d), jnp.float32),
            pltpu.VMEM_SHARED((NP, d), jnp.float32),
            pltpu.SemaphoreType.DMA,
            pltpu.SemaphoreType.DMA,
        ],
        compiler_params=pltpu.CompilerParams(use_tc_tiling_on_sc=False),
    )
    def sc_kernel(gidx_hbm, didx_hbm, zeros_hbm, table_hbm, out_hbm,
                  gidx_v, didx_v, rows_a, rows_b, acc_sh, sem_a, sem_b):
        cid = lax.axis_index("c")
        sid = lax.axis_index("s")
        base = sid * SPS + cid * s0
        steps = lax.select(cid == 0, s0, s1)

        pltpu.sync_copy(zeros_hbm.at[pl.ds(sid * ROWS, ROWS)],
                        acc_sh.at[pl.ds(sid * ROWS, ROWS)])
        pltpu.sync_copy(gidx_hbm.at[pl.ds(base, smax_g)], gidx_v)
        pltpu.sync_copy(didx_hbm.at[pl.ds(base, smax)], didx_v)
        plsc.subcore_barrier()

        if pipelined:
            pltpu.async_copy(table_hbm.at[gidx_v.at[0]], rows_a, sem_a)

            def body(i, carry):
                j = 2 * i
                pltpu.async_copy(table_hbm.at[gidx_v.at[j + 1]],
                                 rows_b, sem_b)
                pltpu.make_async_copy(table_hbm.at[gidx_v.at[j]],
                                      rows_a, sem_a).wait()
                pltpu.sync_copy(rows_a, acc_sh.at[didx_v.at[j]], add=True)
                pltpu.async_copy(table_hbm.at[gidx_v.at[j + 2]],
                                 rows_a, sem_a)
                pltpu.make_async_copy(table_hbm.at[gidx_v.at[j + 1]],
                                      rows_b, sem_b).wait()
                pltpu.sync_copy(rows_b, acc_sh.at[didx_v.at[j + 1]], add=True)
                return carry

            lax.fori_loop(0, steps // 2, body, 0)
            pltpu.make_async_copy(table_hbm.at[gidx_v.at[steps]],
                                  rows_a, sem_a).wait()
        else:
            def body(j, carry):
                pltpu.async_copy(table_hbm.at[gidx_v.at[j]],
                                 rows_a, sem_a).wait()
                pltpu.sync_copy(rows_a, acc_sh.at[didx_v.at[j]], add=True)
                return carry

            lax.fori_loop(0, steps, body, 0)
        plsc.subcore_barrier()

        pltpu.sync_copy(acc_sh.at[pl.ds(sid * ROWS, ROWS)],
                        out_hbm.at[cid, pl.ds(sid * ROWS, ROWS)])

    return sc_kernel


_EXTRA = 128
_sc_scatter_64 = _make_sc_scatter(64, 112)
_sc_scatter_16 = _make_sc_scatter(16, 100, pipelined=True)

_BN = 400
_GRID = N // _BN


def _mm1(x, v1, comb1, loop_w1):
    h = v1.shape[-1]
    return pl.pallas_call(
        _mm1_kernel,
        grid=(_GRID,),
        in_specs=[
            pl.BlockSpec((_BN, x.shape[1]), lambda i: (i, 0)),
            pl.BlockSpec(v1.shape, lambda i: (0, 0, 0)),
            pl.BlockSpec(comb1.shape, lambda i: (0, 0)),
            pl.BlockSpec(loop_w1.shape, lambda i: (0, 0)),
        ],
        out_specs=[
            pl.BlockSpec((R // 2, _BN, 128), lambda i: (0, i, 0)),
            pl.BlockSpec((_BN, h), lambda i: (i, 0)),
        ],
        out_shape=[
            jax.ShapeDtypeStruct((R // 2, N, 128), jnp.float32),
            jax.ShapeDtypeStruct((N, h), jnp.float32),
        ],
    )(x, v1, comb1, loop_w1)


def _mm2(parts, loop1, b1, v2, comb2, loop_w2):
    h = parts.shape[-1]
    c = v2.shape[-1]
    return pl.pallas_call(
        _mm2_kernel,
        grid=(_GRID,),
        in_specs=[
            pl.BlockSpec((NC, _BN, h), lambda i: (0, i, 0)),
            pl.BlockSpec((_BN, h), lambda i: (i, 0)),
            pl.BlockSpec((1, h), lambda i: (0, 0)),
            pl.BlockSpec(v2.shape, lambda i: (0, 0, 0)),
            pl.BlockSpec(comb2.shape, lambda i: (0, 0)),
            pl.BlockSpec(loop_w2.shape, lambda i: (0, 0)),
        ],
        out_specs=[
            pl.BlockSpec((_BN, 128), lambda i: (i, 0)),
            pl.BlockSpec((_BN, c), lambda i: (i, 0)),
        ],
        out_shape=[
            jax.ShapeDtypeStruct((N, 128), jnp.float32),
            jax.ShapeDtypeStruct((N, c), jnp.float32),
        ],
    )(parts, loop1, b1, v2, comb2, loop_w2)


def _final(parts, loop2, b2):
    c = parts.shape[-1]
    return pl.pallas_call(
        _final_kernel,
        grid=(_GRID,),
        in_specs=[
            pl.BlockSpec((NC, _BN, c), lambda i: (0, i, 0)),
            pl.BlockSpec((_BN, c), lambda i: (i, 0)),
            pl.BlockSpec((1, c), lambda i: (0, 0)),
        ],
        out_specs=pl.BlockSpec((1, c), lambda i: (0, 0)),
        out_shape=jax.ShapeDtypeStruct((1, c), jnp.float32),
    )(parts, loop2, b2)


def kernel(in_feat, edge_index, etypes, V1, comb1, loop_w1, b1,
           V2, comb2, loop_w2, b2):
    src = edge_index[0]
    dst = edge_index[1]
    et = etypes.reshape(-1)

    gidx1 = ((et >> 1) * (2 * N) + 2 * src + (et & 1)).astype(jnp.int32)
    gidx2 = (src * R + et).astype(jnp.int32)
    didx = dst.astype(jnp.int32)
    pad = E_PAD - E
    padx = pad + _EXTRA * K
    zpad = jnp.zeros((padx,), jnp.int32)
    gidx1 = jnp.concatenate([gidx1, zpad]).reshape(TOT_ROWS + _EXTRA, K)
    gidx2 = jnp.concatenate([gidx2, zpad]).reshape(TOT_ROWS + _EXTRA, K)
    didx = jnp.concatenate([didx, jnp.full((padx,), N, jnp.int32)])
    didx = didx.reshape(TOT_ROWS + _EXTRA, K)

    h = V1.shape[-1]
    c = V2.shape[-1]
    zeros_h = jnp.zeros((NP, h), jnp.float32)
    zeros_c = jnp.zeros((NP, c), jnp.float32)

    y1, loop1 = _mm1(in_feat, V1, comb1, loop_w1)
    parts1 = _sc_scatter_64(gidx1, didx, zeros_h, y1.reshape(R * N, h))

    y2, loop2 = _mm2(parts1, loop1, b1.reshape(1, h), V2, comb2, loop_w2)
    parts2 = _sc_scatter_16(gidx2, didx, zeros_c, y2.reshape(R * N, c))

    return _final(parts2, loop2, b2.reshape(1, c))


# speedup vs baseline: 1.7624x; 1.0587x over previous
import functools

import jax
import jax.numpy as jnp
from jax import lax
from jax.experimental import pallas as pl
from jax.experimental.pallas import tpu as pltpu
from jax.experimental.pallas import tpu_sc as plsc

N = 10000
E = 320000
R = 8

NC = 2
NS = 16
NW = NC * NS

K = 128
SPS = 160
E_PAD = SPS * NS * K
TOT_ROWS = E_PAD // K
NP = 10112
ROWS = NP // NS


def _mm1_kernel(x_ref, v_ref, comb_ref, loopw_ref, y_ref, loop_ref):
    w = jnp.sum(comb_ref[...][:, :, None, None] * v_ref[...][None], axis=1)
    x = x_ref[...]
    for p in range(R // 2):
        wp = jnp.concatenate([w[2 * p], w[2 * p + 1]], axis=-1)
        y_ref[p] = jnp.dot(x, wp, preferred_element_type=jnp.float32)
    loop_ref[...] = jnp.dot(x, loopw_ref[...], preferred_element_type=jnp.float32)


def _mm2_kernel(p_ref, loop1_ref, b1_ref, v_ref, comb_ref, loopw_ref,
                y_ref, loop_ref):
    h = p_ref[0] + p_ref[1] + loop1_ref[...] + b1_ref[...]
    h = jnp.maximum(h, 0.0)
    w = jnp.sum(comb_ref[...][:, :, None, None] * v_ref[...][None], axis=1)
    wcat = jnp.concatenate([w[r] for r in range(R)], axis=-1)
    y_ref[...] = jnp.dot(h, wcat, preferred_element_type=jnp.float32)
    loop_ref[...] = jnp.dot(h, loopw_ref[...], preferred_element_type=jnp.float32)


def _final_kernel(p_ref, loop2_ref, b2_ref, out_ref):
    i = pl.program_id(0)

    @pl.when(i == 0)
    def _():
        out_ref[...] = jnp.zeros_like(out_ref)

    h = p_ref[0] + p_ref[1] + loop2_ref[...] + b2_ref[...]
    h = jnp.maximum(h, 0.0)
    out_ref[...] += jnp.sum(h, axis=0, keepdims=True) * (1.0 / N)


def _make_sc_scatter(d, s0, pipelined=False, table_in_spmem=False):
    s1 = SPS - s0
    smax = max(s0, s1)
    smax_g = smax + 2 if pipelined else smax
    trows = R * N // NS
    mesh = plsc.VectorSubcoreMesh(core_axis_name="c", subcore_axis_name="s")

    scratch = [
        pltpu.VMEM((smax_g, K), jnp.int32),
        pltpu.VMEM((smax, K), jnp.int32),
        pltpu.VMEM((K, d), jnp.float32),
        pltpu.VMEM((K, d), jnp.float32),
        pltpu.VMEM_SHARED((NP, d), jnp.float32),
        pltpu.SemaphoreType.DMA,
        pltpu.SemaphoreType.DMA,
    ]
    if table_in_spmem:
        scratch.append(pltpu.VMEM_SHARED((R * N, d), jnp.float32))

    @functools.partial(
        pl.kernel,
        mesh=mesh,
        out_type=jax.ShapeDtypeStruct((NC, NP, d), jnp.float32),
        scratch_types=scratch,
        compiler_params=pltpu.CompilerParams(use_tc_tiling_on_sc=False),
    )
    def sc_kernel(gidx_hbm, didx_hbm, zeros_hbm, table_hbm, out_hbm,
                  gidx_v, didx_v, rows_a, rows_b, acc_sh, sem_a, sem_b,
                  *maybe_tab):
        cid = lax.axis_index("c")
        sid = lax.axis_index("s")
        base = sid * SPS + cid * s0
        steps = lax.select(cid == 0, s0, s1)

        pltpu.sync_copy(zeros_hbm.at[pl.ds(sid * ROWS, ROWS)],
                        acc_sh.at[pl.ds(sid * ROWS, ROWS)])
        if table_in_spmem:
            pltpu.sync_copy(table_hbm.at[pl.ds(sid * trows, trows)],
                            maybe_tab[0].at[pl.ds(sid * trows, trows)])
            table = maybe_tab[0]
        else:
            table = table_hbm
        pltpu.sync_copy(gidx_hbm.at[pl.ds(base, smax_g)], gidx_v)
        pltpu.sync_copy(didx_hbm.at[pl.ds(base, smax)], didx_v)
        plsc.subcore_barrier()

        if pipelined:
            pltpu.async_copy(table.at[gidx_v.at[0]], rows_a, sem_a)

            def body(i, carry):
                j = 2 * i
                pltpu.async_copy(table.at[gidx_v.at[j + 1]],
                                 rows_b, sem_b)
                pltpu.make_async_copy(table.at[gidx_v.at[j]],
                                      rows_a, sem_a).wait()
                pltpu.sync_copy(rows_a, acc_sh.at[didx_v.at[j]], add=True)
                pltpu.async_copy(table.at[gidx_v.at[j + 2]],
                                 rows_a, sem_a)
                pltpu.make_async_copy(table.at[gidx_v.at[j + 1]],
                                      rows_b, sem_b).wait()
                pltpu.sync_copy(rows_b, acc_sh.at[didx_v.at[j + 1]], add=True)
                return carry

            lax.fori_loop(0, steps // 2, body, 0)
            pltpu.make_async_copy(table.at[gidx_v.at[steps]],
                                  rows_a, sem_a).wait()
        else:
            def body(j, carry):
                pltpu.async_copy(table.at[gidx_v.at[j]],
                                 rows_a, sem_a).wait()
                pltpu.sync_copy(rows_a, acc_sh.at[didx_v.at[j]], add=True)
                return carry

            lax.fori_loop(0, steps, body, 0)
        plsc.subcore_barrier()

        pltpu.sync_copy(acc_sh.at[pl.ds(sid * ROWS, ROWS)],
                        out_hbm.at[cid, pl.ds(sid * ROWS, ROWS)])

    return sc_kernel


_EXTRA = 128
_sc_scatter_64 = _make_sc_scatter(64, 112)
_sc_scatter_16 = _make_sc_scatter(16, 100, pipelined=True, table_in_spmem=True)

_BN = 400
_GRID = N // _BN


def _mm1(x, v1, comb1, loop_w1):
    h = v1.shape[-1]
    return pl.pallas_call(
        _mm1_kernel,
        grid=(_GRID,),
        in_specs=[
            pl.BlockSpec((_BN, x.shape[1]), lambda i: (i, 0)),
            pl.BlockSpec(v1.shape, lambda i: (0, 0, 0)),
            pl.BlockSpec(comb1.shape, lambda i: (0, 0)),
            pl.BlockSpec(loop_w1.shape, lambda i: (0, 0)),
        ],
        out_specs=[
            pl.BlockSpec((R // 2, _BN, 128), lambda i: (0, i, 0)),
            pl.BlockSpec((_BN, h), lambda i: (i, 0)),
        ],
        out_shape=[
            jax.ShapeDtypeStruct((R // 2, N, 128), jnp.float32),
            jax.ShapeDtypeStruct((N, h), jnp.float32),
        ],
    )(x, v1, comb1, loop_w1)


def _mm2(parts, loop1, b1, v2, comb2, loop_w2):
    h = parts.shape[-1]
    c = v2.shape[-1]
    return pl.pallas_call(
        _mm2_kernel,
        grid=(_GRID,),
        in_specs=[
            pl.BlockSpec((NC, _BN, h), lambda i: (0, i, 0)),
            pl.BlockSpec((_BN, h), lambda i: (i, 0)),
            pl.BlockSpec((1, h), lambda i: (0, 0)),
            pl.BlockSpec(v2.shape, lambda i: (0, 0, 0)),
            pl.BlockSpec(comb2.shape, lambda i: (0, 0)),
            pl.BlockSpec(loop_w2.shape, lambda i: (0, 0)),
        ],
        out_specs=[
            pl.BlockSpec((_BN, 128), lambda i: (i, 0)),
            pl.BlockSpec((_BN, c), lambda i: (i, 0)),
        ],
        out_shape=[
            jax.ShapeDtypeStruct((N, 128), jnp.float32),
            jax.ShapeDtypeStruct((N, c), jnp.float32),
        ],
    )(parts, loop1, b1, v2, comb2, loop_w2)


def _final(parts, loop2, b2):
    c = parts.shape[-1]
    return pl.pallas_call(
        _final_kernel,
        grid=(_GRID,),
        in_specs=[
            pl.BlockSpec((NC, _BN, c), lambda i: (0, i, 0)),
            pl.BlockSpec((_BN, c), lambda i: (i, 0)),
            pl.BlockSpec((1, c), lambda i: (0, 0)),
        ],
        out_specs=pl.BlockSpec((1, c), lambda i: (0, 0)),
        out_shape=jax.ShapeDtypeStruct((1, c), jnp.float32),
    )(parts, loop2, b2)


def kernel(in_feat, edge_index, etypes, V1, comb1, loop_w1, b1,
           V2, comb2, loop_w2, b2):
    src = edge_index[0]
    dst = edge_index[1]
    et = etypes.reshape(-1)

    gidx1 = ((et >> 1) * (2 * N) + 2 * src + (et & 1)).astype(jnp.int32)
    gidx2 = (src * R + et).astype(jnp.int32)
    didx = dst.astype(jnp.int32)
    pad = E_PAD - E
    padx = pad + _EXTRA * K
    zpad = jnp.zeros((padx,), jnp.int32)
    gidx1 = jnp.concatenate([gidx1, zpad]).reshape(TOT_ROWS + _EXTRA, K)
    gidx2 = jnp.concatenate([gidx2, zpad]).reshape(TOT_ROWS + _EXTRA, K)
    didx = jnp.concatenate([didx, jnp.full((padx,), N, jnp.int32)])
    didx = didx.reshape(TOT_ROWS + _EXTRA, K)

    h = V1.shape[-1]
    c = V2.shape[-1]
    zeros_h = jnp.zeros((NP, h), jnp.float32)
    zeros_c = jnp.zeros((NP, c), jnp.float32)

    y1, loop1 = _mm1(in_feat, V1, comb1, loop_w1)
    parts1 = _sc_scatter_64(gidx1, didx, zeros_h, y1.reshape(R * N, h))

    y2, loop2 = _mm2(parts1, loop1, b1.reshape(1, h), V2, comb2, loop_w2)
    parts2 = _sc_scatter_16(gidx2, didx, zeros_c, y2.reshape(R * N, c))

    return _final(parts2, loop2, b2.reshape(1, c))


# speedup vs baseline: 1.7806x; 1.0103x over previous
import functools

import jax
import jax.numpy as jnp
from jax import lax
from jax.experimental import pallas as pl
from jax.experimental.pallas import tpu as pltpu
from jax.experimental.pallas import tpu_sc as plsc

N = 10000
E = 320000
R = 8

NC = 2
NS = 16
NW = NC * NS

K = 128
SPS = 160
E_PAD = SPS * NS * K
TOT_ROWS = E_PAD // K
NP = 10112
ROWS = NP // NS


def _mm1_kernel(x_ref, v_ref, comb_ref, loopw_ref, y_ref, loop_ref):
    w = jnp.sum(comb_ref[...][:, :, None, None] * v_ref[...][None], axis=1)
    x = x_ref[...]
    for p in range(R // 2):
        wp = jnp.concatenate([w[2 * p], w[2 * p + 1]], axis=-1)
        y_ref[p] = jnp.dot(x, wp, preferred_element_type=jnp.float32)
    loop_ref[...] = jnp.dot(x, loopw_ref[...], preferred_element_type=jnp.float32)


def _mm2_kernel(p_ref, loop1_ref, b1_ref, v_ref, comb_ref, loopw_ref,
                y_ref, loop_ref):
    h = p_ref[0] + p_ref[1] + loop1_ref[...] + b1_ref[...]
    h = jnp.maximum(h, 0.0)
    w = jnp.sum(comb_ref[...][:, :, None, None] * v_ref[...][None], axis=1)
    wcat = jnp.concatenate([w[r] for r in range(R)], axis=-1)
    y_ref[...] = jnp.dot(h, wcat, preferred_element_type=jnp.float32)
    loop_ref[...] = jnp.dot(h, loopw_ref[...], preferred_element_type=jnp.float32)


def _final_kernel(p_ref, loop2_ref, b2_ref, out_ref):
    i = pl.program_id(0)

    @pl.when(i == 0)
    def _():
        out_ref[...] = jnp.zeros_like(out_ref)

    h = p_ref[0] + p_ref[1] + loop2_ref[...] + b2_ref[...]
    h = jnp.maximum(h, 0.0)
    out_ref[...] += jnp.sum(h, axis=0, keepdims=True) * (1.0 / N)


def _make_sc_scatter(d, s0, pipelined=False, table_in_spmem=False):
    s1 = SPS - s0
    smax = max(s0, s1)
    smax_g = smax + 2 if pipelined else smax
    trows = R * N // NS
    mesh = plsc.VectorSubcoreMesh(core_axis_name="c", subcore_axis_name="s")

    scratch = [
        pltpu.VMEM((smax_g, K), jnp.int32),
        pltpu.VMEM((smax, K), jnp.int32),
        pltpu.VMEM((K, d), jnp.float32),
        pltpu.VMEM((K, d), jnp.float32),
        pltpu.VMEM_SHARED((NP, d), jnp.float32),
        pltpu.SemaphoreType.DMA,
        pltpu.SemaphoreType.DMA,
    ]
    if table_in_spmem:
        scratch.append(pltpu.VMEM_SHARED((R * N, d), jnp.float32))

    @functools.partial(
        pl.kernel,
        mesh=mesh,
        out_type=jax.ShapeDtypeStruct((NC, NP, d), jnp.float32),
        scratch_types=scratch,
        compiler_params=pltpu.CompilerParams(use_tc_tiling_on_sc=False),
    )
    def sc_kernel(gidx_hbm, didx_hbm, zeros_hbm, table_hbm, out_hbm,
                  gidx_v, didx_v, rows_a, rows_b, acc_sh, sem_a, sem_b,
                  *maybe_tab):
        cid = lax.axis_index("c")
        sid = lax.axis_index("s")
        base = sid * SPS + cid * s0
        steps = lax.select(cid == 0, s0, s1)

        pltpu.sync_copy(zeros_hbm.at[pl.ds(sid * ROWS, ROWS)],
                        acc_sh.at[pl.ds(sid * ROWS, ROWS)])
        if table_in_spmem:
            pltpu.sync_copy(table_hbm.at[pl.ds(sid * trows, trows)],
                            maybe_tab[0].at[pl.ds(sid * trows, trows)])
            table = maybe_tab[0]
        else:
            table = table_hbm
        pltpu.sync_copy(gidx_hbm.at[pl.ds(base, smax_g)], gidx_v)
        pltpu.sync_copy(didx_hbm.at[pl.ds(base, smax)], didx_v)
        plsc.subcore_barrier()

        if pipelined:
            pltpu.async_copy(table.at[gidx_v.at[0]], rows_a, sem_a)

            def body(i, carry):
                j = 2 * i
                pltpu.async_copy(table.at[gidx_v.at[j + 1]],
                                 rows_b, sem_b)
                pltpu.make_async_copy(table.at[gidx_v.at[j]],
                                      rows_a, sem_a).wait()
                pltpu.sync_copy(rows_a, acc_sh.at[didx_v.at[j]], add=True)
                pltpu.async_copy(table.at[gidx_v.at[j + 2]],
                                 rows_a, sem_a)
                pltpu.make_async_copy(table.at[gidx_v.at[j + 1]],
                                      rows_b, sem_b).wait()
                pltpu.sync_copy(rows_b, acc_sh.at[didx_v.at[j + 1]], add=True)
                return carry

            lax.fori_loop(0, steps // 2, body, 0)
            pltpu.make_async_copy(table.at[gidx_v.at[steps]],
                                  rows_a, sem_a).wait()
        else:
            def body(j, carry):
                pltpu.async_copy(table.at[gidx_v.at[j]],
                                 rows_a, sem_a).wait()
                pltpu.sync_copy(rows_a, acc_sh.at[didx_v.at[j]], add=True)
                return carry

            lax.fori_loop(0, steps, body, 0)
        plsc.subcore_barrier()

        pltpu.sync_copy(acc_sh.at[pl.ds(sid * ROWS, ROWS)],
                        out_hbm.at[cid, pl.ds(sid * ROWS, ROWS)])

    return sc_kernel


_EXTRA = 128
_sc_scatter_64 = _make_sc_scatter(64, 111)
_sc_scatter_16 = _make_sc_scatter(16, 92, pipelined=True, table_in_spmem=True)

_BN = 400
_GRID = N // _BN


def _mm1(x, v1, comb1, loop_w1):
    h = v1.shape[-1]
    return pl.pallas_call(
        _mm1_kernel,
        grid=(_GRID,),
        in_specs=[
            pl.BlockSpec((_BN, x.shape[1]), lambda i: (i, 0)),
            pl.BlockSpec(v1.shape, lambda i: (0, 0, 0)),
            pl.BlockSpec(comb1.shape, lambda i: (0, 0)),
            pl.BlockSpec(loop_w1.shape, lambda i: (0, 0)),
        ],
        out_specs=[
            pl.BlockSpec((R // 2, _BN, 128), lambda i: (0, i, 0)),
            pl.BlockSpec((_BN, h), lambda i: (i, 0)),
        ],
        out_shape=[
            jax.ShapeDtypeStruct((R // 2, N, 128), jnp.float32),
            jax.ShapeDtypeStruct((N, h), jnp.float32),
        ],
    )(x, v1, comb1, loop_w1)


def _mm2(parts, loop1, b1, v2, comb2, loop_w2):
    h = parts.shape[-1]
    c = v2.shape[-1]
    return pl.pallas_call(
        _mm2_kernel,
        grid=(_GRID,),
        in_specs=[
            pl.BlockSpec((NC, _BN, h), lambda i: (0, i, 0)),
            pl.BlockSpec((_BN, h), lambda i: (i, 0)),
            pl.BlockSpec((1, h), lambda i: (0, 0)),
            pl.BlockSpec(v2.shape, lambda i: (0, 0, 0)),
            pl.BlockSpec(comb2.shape, lambda i: (0, 0)),
            pl.BlockSpec(loop_w2.shape, lambda i: (0, 0)),
        ],
        out_specs=[
            pl.BlockSpec((_BN, 128), lambda i: (i, 0)),
            pl.BlockSpec((_BN, c), lambda i: (i, 0)),
        ],
        out_shape=[
            jax.ShapeDtypeStruct((N, 128), jnp.float32),
            jax.ShapeDtypeStruct((N, c), jnp.float32),
        ],
    )(parts, loop1, b1, v2, comb2, loop_w2)


def _final(parts, loop2, b2):
    c = parts.shape[-1]
    return pl.pallas_call(
        _final_kernel,
        grid=(_GRID,),
        in_specs=[
            pl.BlockSpec((NC, _BN, c), lambda i: (0, i, 0)),
            pl.BlockSpec((_BN, c), lambda i: (i, 0)),
            pl.BlockSpec((1, c), lambda i: (0, 0)),
        ],
        out_specs=pl.BlockSpec((1, c), lambda i: (0, 0)),
        out_shape=jax.ShapeDtypeStruct((1, c), jnp.float32),
    )(parts, loop2, b2)


def kernel(in_feat, edge_index, etypes, V1, comb1, loop_w1, b1,
           V2, comb2, loop_w2, b2):
    src = edge_index[0]
    dst = edge_index[1]
    et = etypes.reshape(-1)

    gidx1 = ((et >> 1) * (2 * N) + 2 * src + (et & 1)).astype(jnp.int32)
    gidx2 = (src * R + et).astype(jnp.int32)
    didx = dst.astype(jnp.int32)
    pad = E_PAD - E
    padx = pad + _EXTRA * K
    zpad = jnp.zeros((padx,), jnp.int32)
    gidx1 = jnp.concatenate([gidx1, zpad]).reshape(TOT_ROWS + _EXTRA, K)
    gidx2 = jnp.concatenate([gidx2, zpad]).reshape(TOT_ROWS + _EXTRA, K)
    didx = jnp.concatenate([didx, jnp.full((padx,), N, jnp.int32)])
    didx = didx.reshape(TOT_ROWS + _EXTRA, K)

    h = V1.shape[-1]
    c = V2.shape[-1]
    zeros_h = jnp.zeros((NP, h), jnp.float32)
    zeros_c = jnp.zeros((NP, c), jnp.float32)

    y1, loop1 = _mm1(in_feat, V1, comb1, loop_w1)
    parts1 = _sc_scatter_64(gidx1, didx, zeros_h, y1.reshape(R * N, h))

    y2, loop2 = _mm2(parts1, loop1, b1.reshape(1, h), V2, comb2, loop_w2)
    parts2 = _sc_scatter_16(gidx2, didx, zeros_c, y2.reshape(R * N, c))

    return _final(parts2, loop2, b2.reshape(1, c))
